# baseline (speedup 1.0000x reference)
import functools

import jax
import jax.numpy as jnp
from jax import lax
from jax.experimental import pallas as pl
from jax.experimental.pallas import tpu as pltpu
from jax.experimental.pallas import tpu_sc as plsc

N = 10000
D = 128
E = 320000
NC = 2
NS = 16
NW = NC * NS
B = 128
CPT = 80
EPAD = NW * CPT * B
NPAD = 10240
RPT = NPAD // NS
ZR = RPT // 4
DW = 16
PAD_ROW = N + 8

@functools.cache
def _make_deg_sc():
    return pl.kernel(
        _deg_sc_body,
        out_type=jax.ShapeDtypeStruct((NC, NPAD, DW), jnp.float32),
        mesh=plsc.VectorSubcoreMesh(
            core_axis_name="c", subcore_axis_name="s",
            num_cores=NC, num_subcores=NS),
        scratch_types=[
            pltpu.VMEM((CPT, B), jnp.int32),
            pltpu.VMEM((B, DW), jnp.float32),
            pltpu.VMEM((RPT, DW), jnp.float32),
            pltpu.VMEM_SHARED((NPAD, DW), jnp.float32),
        ],
        compiler_params=pltpu.CompilerParams(use_tc_tiling_on_sc=False),
    )


def _deg_sc_body(dst_hbm, ones_hbm, zeros_hbm, out_hbm, dst_v, ones_v,
                 stage_v, acc):
    c = lax.axis_index("c")
    s = lax.axis_index("s")
    wid = c * NS + s
    row0 = s * RPT

    pltpu.sync_copy(dst_hbm.at[wid], dst_v)
    pltpu.sync_copy(ones_hbm, ones_v)
    pltpu.sync_copy(zeros_hbm, stage_v)
    pltpu.sync_copy(stage_v, acc.at[pl.ds(row0, RPT)])
    plsc.subcore_barrier()

    def body(j, carry):
        pltpu.sync_copy(ones_v, acc.at[dst_v.at[j]], add=True)
        return carry

    lax.fori_loop(0, CPT, body, 0)
    plsc.subcore_barrier()

    pltpu.sync_copy(acc.at[pl.ds(row0, RPT)], stage_v)
    pltpu.sync_copy(stage_v, out_hbm.at[c, pl.ds(row0, RPT)])


CPT_F = 160
GRP = 16
TOT_CH = NS * CPT_F
ZR2 = 40


@functools.cache
def _make_agg_sc():
    return pl.kernel(
        _agg_sc_body,
        out_type=jax.ShapeDtypeStruct((NPAD, D), jnp.float32),
        mesh=plsc.VectorSubcoreMesh(
            core_axis_name="c", subcore_axis_name="s",
            num_cores=NC, num_subcores=NS),
        scratch_types=[
            [pltpu.VMEM((GRP, B), jnp.int32) for _ in range(2)],
            [pltpu.VMEM((GRP, B), jnp.int32) for _ in range(2)],
            [pltpu.VMEM((B, D), jnp.float32) for _ in range(2)],
            pltpu.VMEM((ZR2, D), jnp.float32),
            pltpu.VMEM_SHARED((NPAD, D), jnp.float32),
            [pltpu.SemaphoreType.DMA for _ in range(2)],
            [pltpu.SemaphoreType.DMA for _ in range(2)],
            pltpu.SemaphoreType.DMA,
        ],
        compiler_params=pltpu.CompilerParams(use_tc_tiling_on_sc=False),
    )


def _agg_sc_body(h_hbm, src_hbm, dst_hbm, zeros_hbm, out_hbm,
                 srcg, dstg, bufs, stage_v, acc, semg, sems, semi):
    c = lax.axis_index("c")
    s = lax.axis_index("s")
    row0 = s * RPT

    def chunk_step(slot, k, refill_slot, refill_k, do_refill):
        b = k % 2
        pltpu.make_async_copy(
            h_hbm.at[srcg[slot].at[k]], bufs[b], semg[b]).wait()
        pltpu.async_copy(bufs[b], acc.at[dstg[slot].at[k]], sems[b],
                         add=True)
        if do_refill:
            pltpu.make_async_copy(
                bufs[b], acc.at[dstg[slot].at[k]], sems[b]).wait()
            pltpu.async_copy(h_hbm.at[srcg[refill_slot].at[refill_k]],
                             bufs[b], semg[b])

    def run(cpt, base):
        ngrp = cpt // GRP
        pltpu.sync_copy(src_hbm.at[pl.ds(base, GRP)], srcg[0])
        pltpu.sync_copy(dst_hbm.at[pl.ds(base, GRP)], dstg[0])
        for b in range(2):
            pltpu.async_copy(h_hbm.at[srcg[0].at[b]], bufs[b], semg[b])

        def group(i2, slot):
            nxt = base + (i2 + 1) * GRP
            pltpu.async_copy(src_hbm.at[pl.ds(nxt, GRP)], srcg[slot ^ 1],
                             semi)
            pltpu.async_copy(dst_hbm.at[pl.ds(nxt, GRP)], dstg[slot ^ 1],
                             semi)
            for k in range(GRP):
                if k == GRP - 2:
                    pltpu.make_async_copy(
                        src_hbm.at[pl.ds(nxt, GRP)], srcg[slot ^ 1],
                        semi).wait()
                    pltpu.make_async_copy(
                        dst_hbm.at[pl.ds(nxt, GRP)], dstg[slot ^ 1],
                        semi).wait()
                if k < GRP - 2:
                    chunk_step(slot, k, slot, k + 2, True)
                else:
                    chunk_step(slot, k, slot ^ 1, k + 2 - GRP, True)

        def pair(i, carry):
            group(2 * i, 0)
            group(2 * i + 1, 1)
            return carry

        lax.fori_loop(0, (ngrp - 2) // 2, pair, 0)
        group(ngrp - 2, 0)
        for k in range(GRP):
            chunk_step(1, k, 1, k + 2, k < GRP - 2)
        for b in range(2):
            pltpu.make_async_copy(bufs[b], acc.at[dstg[1].at[GRP - 2 + b]],
                                  sems[b]).wait()

    @pl.when(c == 1)
    def _():
        pltpu.sync_copy(zeros_hbm, stage_v)
        for k in range(RPT // ZR2):
            pltpu.sync_copy(stage_v, acc.at[pl.ds(row0 + k * ZR2, ZR2)])
        plsc.subcore_barrier()
        run(CPT_F, s * CPT_F)
        plsc.subcore_barrier()
        for k in range(RPT // ZR2):
            pltpu.sync_copy(acc.at[pl.ds(row0 + k * ZR2, ZR2)], stage_v)
            pltpu.sync_copy(stage_v, out_hbm.at[pl.ds(row0 + k * ZR2, ZR2)])


def _dinv_block(d0_ref, d1_ref):
    deg = d0_ref[:, 0:1] + d1_ref[:, 0:1] + 1.0
    return lax.rsqrt(deg)


def _tc_a_body(x_ref, w_ref, d0_ref, d1_ref, o_ref):
    dinv = _dinv_block(d0_ref, d1_ref)
    h = jnp.dot(x_ref[...], w_ref[...], preferred_element_type=jnp.float32)
    o_ref[...] = h * dinv


def _tc_b_body(p_ref, hp_ref, b_ref, w_ref, d0_ref, d1_ref, o_ref):
    dinv = _dinv_block(d0_ref, d1_ref)
    z = (p_ref[...] + hp_ref[...]) * dinv + b_ref[...]
    y = jnp.maximum(z, 0.0)
    h = jnp.dot(y, w_ref[...], preferred_element_type=jnp.float32)
    o_ref[...] = h * dinv


def _tc_c_body(p_ref, hp_ref, b_ref, w_ref, bl_ref, d0_ref, d1_ref,
               o_ref):
    dinv = _dinv_block(d0_ref, d1_ref)
    z = (p_ref[...] + hp_ref[...]) * dinv + b_ref[...]
    y = jnp.maximum(z, 0.0)
    h = jnp.dot(y, w_ref[...], preferred_element_type=jnp.float32) + bl_ref[...]
    o_ref[...] = jnp.maximum(h, 0.0)


_TCR = 2000


def _rows_spec(width=D):
    return pl.BlockSpec((_TCR, width), lambda i: (i, 0))


def _full_spec(shape):
    return pl.BlockSpec(shape, lambda i: (0, 0))


def _tc_a(x, W1, d0, d1):
    return pl.pallas_call(
        _tc_a_body,
        grid=(N // _TCR,),
        in_specs=[_rows_spec(), _full_spec((D, D)), _rows_spec(DW),
                  _rows_spec(DW)],
        out_specs=_rows_spec(),
        out_shape=jax.ShapeDtypeStruct((N, D), jnp.float32),
    )(x, W1, d0, d1)


def _tc_b(p, hp, b, W, d0, d1):
    return pl.pallas_call(
        _tc_b_body,
        grid=(N // _TCR,),
        in_specs=[_rows_spec(), _rows_spec(),
                  _full_spec((1, D)), _full_spec((D, D)), _rows_spec(DW),
                  _rows_spec(DW)],
        out_specs=_rows_spec(),
        out_shape=jax.ShapeDtypeStruct((N, D), jnp.float32),
    )(p, hp, b, W, d0, d1)


def _tc_c(p, hp, b, W, bl, d0, d1):
    return pl.pallas_call(
        _tc_c_body,
        grid=(N // _TCR,),
        in_specs=[_rows_spec(), _rows_spec(),
                  _full_spec((1, D)), _full_spec((D, D)), _full_spec((1, D)),
                  _rows_spec(DW), _rows_spec(DW)],
        out_specs=_rows_spec(),
        out_shape=jax.ShapeDtypeStruct((N, D), jnp.float32),
    )(p, hp, b, W, bl, d0, d1)


@jax.jit
def kernel(x, edge_index, W1, b1, W2, b2, Wl, bl):
    src = edge_index[0].astype(jnp.int32)
    dst = edge_index[1].astype(jnp.int32)
    pad = EPAD - E
    srcp = jnp.concatenate([src, jnp.zeros((pad,), jnp.int32)])
    dstp = jnp.concatenate([dst, jnp.full((pad,), PAD_ROW, jnp.int32)])
    srcp = srcp.reshape(NW, CPT, B)
    dstp = dstp.reshape(NW, CPT, B)

    ones_w = jnp.ones((B, DW), jnp.float32)
    zeros_w = jnp.zeros((RPT, DW), jnp.float32)
    zeros_d = jnp.zeros((ZR2, D), jnp.float32)
    srcp2 = srcp.reshape(TOT_CH, B)
    dstp2 = dstp.reshape(TOT_CH, B)

    degp = _make_deg_sc()(dstp, ones_w, zeros_w)
    d0 = degp[0, :N, :]
    d1 = degp[1, :N, :]

    def agg(hp):
        return _make_agg_sc()(hp, srcp2, dstp2, zeros_d)[:N]

    hp1 = _tc_a(x, W1, d0, d1)
    p1x = agg(hp1)
    hp2 = _tc_b(p1x, hp1, b1.reshape(1, D), W2, d0, d1)
    p2x = agg(hp2)
    out = _tc_c(p2x, hp2, b2.reshape(1, D), Wl,
                bl.reshape(1, D), d0, d1)
    return out


# speedup vs baseline: 1.3515x; 1.3515x over previous
import functools

import jax
import jax.numpy as jnp
from jax import lax
from jax.experimental import pallas as pl
from jax.experimental.pallas import tpu as pltpu
from jax.experimental.pallas import tpu_sc as plsc

N = 10000
D = 128
E = 320000
NC = 2
NS = 16
NW = NC * NS
B = 128
CPT = 80
EPAD = NW * CPT * B
NPAD = 10240
RPT = NPAD // NS
ZR = RPT // 4
DW = 16
PAD_ROW = N + 8

@functools.cache
def _make_deg_sc():
    return pl.kernel(
        _deg_sc_body,
        out_type=jax.ShapeDtypeStruct((NC, NPAD, DW), jnp.float32),
        mesh=plsc.VectorSubcoreMesh(
            core_axis_name="c", subcore_axis_name="s",
            num_cores=NC, num_subcores=NS),
        scratch_types=[
            pltpu.VMEM((CPT, B), jnp.int32),
            pltpu.VMEM((B, DW), jnp.float32),
            pltpu.VMEM((RPT, DW), jnp.float32),
            pltpu.VMEM_SHARED((NPAD, DW), jnp.float32),
        ],
        compiler_params=pltpu.CompilerParams(use_tc_tiling_on_sc=False),
    )


def _deg_sc_body(dst_hbm, ones_hbm, zeros_hbm, out_hbm, dst_v, ones_v,
                 stage_v, acc):
    c = lax.axis_index("c")
    s = lax.axis_index("s")
    wid = c * NS + s
    row0 = s * RPT

    pltpu.sync_copy(dst_hbm.at[wid], dst_v)
    pltpu.sync_copy(ones_hbm, ones_v)
    pltpu.sync_copy(zeros_hbm, stage_v)
    pltpu.sync_copy(stage_v, acc.at[pl.ds(row0, RPT)])
    plsc.subcore_barrier()

    def body(j, carry):
        pltpu.sync_copy(ones_v, acc.at[dst_v.at[j]], add=True)
        return carry

    lax.fori_loop(0, CPT, body, 0)
    plsc.subcore_barrier()

    pltpu.sync_copy(acc.at[pl.ds(row0, RPT)], stage_v)
    pltpu.sync_copy(stage_v, out_hbm.at[c, pl.ds(row0, RPT)])


CPT_F = 144
CPT_S = 16
GRP = 16
TOT_CH = NS * (CPT_F + CPT_S)
ZR2 = 40


@functools.cache
def _make_agg_sc():
    return pl.kernel(
        _agg_sc_body,
        out_type=jax.ShapeDtypeStruct((NC, NPAD, D), jnp.float32),
        mesh=plsc.VectorSubcoreMesh(
            core_axis_name="c", subcore_axis_name="s",
            num_cores=NC, num_subcores=NS),
        scratch_types=[
            [pltpu.VMEM((GRP, B), jnp.int32) for _ in range(2)],
            [pltpu.VMEM((GRP, B), jnp.int32) for _ in range(2)],
            [pltpu.VMEM((B, D), jnp.float32) for _ in range(2)],
            pltpu.VMEM((ZR2, D), jnp.float32),
            pltpu.VMEM_SHARED((NPAD, D), jnp.float32),
            [pltpu.SemaphoreType.DMA for _ in range(2)],
            [pltpu.SemaphoreType.DMA for _ in range(2)],
            pltpu.SemaphoreType.DMA,
        ],
        compiler_params=pltpu.CompilerParams(use_tc_tiling_on_sc=False),
    )


def _agg_sc_body(h_hbm, src_hbm, dst_hbm, zeros_hbm, out_hbm,
                 srcg, dstg, bufs, stage_v, acc, semg, sems, semi):
    c = lax.axis_index("c")
    s = lax.axis_index("s")
    row0 = s * RPT

    def chunk_step(slot, k, refill_slot, refill_k, do_refill):
        b = k % 2
        pltpu.make_async_copy(
            h_hbm.at[srcg[slot].at[k]], bufs[b], semg[b]).wait()
        pltpu.async_copy(bufs[b], acc.at[dstg[slot].at[k]], sems[b],
                         add=True)
        if do_refill:
            pltpu.make_async_copy(
                bufs[b], acc.at[dstg[slot].at[k]], sems[b]).wait()
            pltpu.async_copy(h_hbm.at[srcg[refill_slot].at[refill_k]],
                             bufs[b], semg[b])

    def run(cpt, base):
        ngrp = cpt // GRP
        pltpu.sync_copy(src_hbm.at[pl.ds(base, GRP)], srcg[0])
        pltpu.sync_copy(dst_hbm.at[pl.ds(base, GRP)], dstg[0])
        for b in range(2):
            pltpu.async_copy(h_hbm.at[srcg[0].at[b]], bufs[b], semg[b])

        def group(i2, slot):
            nxt = base + (i2 + 1) * GRP
            pltpu.async_copy(src_hbm.at[pl.ds(nxt, GRP)], srcg[slot ^ 1],
                             semi)
            pltpu.async_copy(dst_hbm.at[pl.ds(nxt, GRP)], dstg[slot ^ 1],
                             semi)
            for k in range(GRP):
                if k == GRP - 2:
                    pltpu.make_async_copy(
                        src_hbm.at[pl.ds(nxt, GRP)], srcg[slot ^ 1],
                        semi).wait()
                    pltpu.make_async_copy(
                        dst_hbm.at[pl.ds(nxt, GRP)], dstg[slot ^ 1],
                        semi).wait()
                if k < GRP - 2:
                    chunk_step(slot, k, slot, k + 2, True)
                else:
                    chunk_step(slot, k, slot ^ 1, k + 2 - GRP, True)

        def pair(i, carry):
            group(2 * i, 0)
            group(2 * i + 1, 1)
            return carry

        if ngrp > 1:
            lax.fori_loop(0, (ngrp - 1) // 2, pair, 0)
        for k in range(GRP):
            chunk_step(0, k, 0, k + 2, k < GRP - 2)
        for b in range(2):
            pltpu.make_async_copy(bufs[b], acc.at[dstg[0].at[GRP - 2 + b]],
                                  sems[b]).wait()

    pltpu.sync_copy(zeros_hbm, stage_v)
    for k in range(RPT // ZR2):
        pltpu.sync_copy(stage_v, acc.at[pl.ds(row0 + k * ZR2, ZR2)])
    plsc.subcore_barrier()

    @pl.when(c == 1)
    def _():
        run(CPT_F, s * CPT_F)

    @pl.when(c == 0)
    def _():
        run(CPT_S, NS * CPT_F + s * CPT_S)

    plsc.subcore_barrier()
    for k in range(RPT // ZR2):
        pltpu.sync_copy(acc.at[pl.ds(row0 + k * ZR2, ZR2)], stage_v)
        pltpu.sync_copy(stage_v, out_hbm.at[c, pl.ds(row0 + k * ZR2, ZR2)])


def _dinv_block(d0_ref, d1_ref):
    deg = d0_ref[:, 0:1] + d1_ref[:, 0:1] + 1.0
    return lax.rsqrt(deg)


def _tc_a_body(x_ref, w_ref, d0_ref, d1_ref, o_ref):
    dinv = _dinv_block(d0_ref, d1_ref)
    h = jnp.dot(x_ref[...], w_ref[...], preferred_element_type=jnp.float32)
    o_ref[...] = h * dinv


def _tc_b_body(p_ref, q_ref, hp_ref, b_ref, w_ref, d0_ref, d1_ref, o_ref):
    dinv = _dinv_block(d0_ref, d1_ref)
    z = (p_ref[...] + q_ref[...] + hp_ref[...]) * dinv + b_ref[...]
    y = jnp.maximum(z, 0.0)
    h = jnp.dot(y, w_ref[...], preferred_element_type=jnp.float32)
    o_ref[...] = h * dinv


def _tc_c_body(p_ref, q_ref, hp_ref, b_ref, w_ref, bl_ref, d0_ref, d1_ref,
               o_ref):
    dinv = _dinv_block(d0_ref, d1_ref)
    z = (p_ref[...] + q_ref[...] + hp_ref[...]) * dinv + b_ref[...]
    y = jnp.maximum(z, 0.0)
    h = jnp.dot(y, w_ref[...], preferred_element_type=jnp.float32) + bl_ref[...]
    o_ref[...] = jnp.maximum(h, 0.0)


_TCR = 2000


def _rows_spec(width=D):
    return pl.BlockSpec((_TCR, width), lambda i: (i, 0))


def _full_spec(shape):
    return pl.BlockSpec(shape, lambda i: (0, 0))


def _tc_a(x, W1, d0, d1):
    return pl.pallas_call(
        _tc_a_body,
        grid=(N // _TCR,),
        in_specs=[_rows_spec(), _full_spec((D, D)), _rows_spec(DW),
                  _rows_spec(DW)],
        out_specs=_rows_spec(),
        out_shape=jax.ShapeDtypeStruct((N, D), jnp.float32),
    )(x, W1, d0, d1)


def _tc_b(p, q, hp, b, W, d0, d1):
    return pl.pallas_call(
        _tc_b_body,
        grid=(N // _TCR,),
        in_specs=[_rows_spec(), _rows_spec(), _rows_spec(),
                  _full_spec((1, D)), _full_spec((D, D)), _rows_spec(DW),
                  _rows_spec(DW)],
        out_specs=_rows_spec(),
        out_shape=jax.ShapeDtypeStruct((N, D), jnp.float32),
    )(p, q, hp, b, W, d0, d1)


def _tc_c(p, q, hp, b, W, bl, d0, d1):
    return pl.pallas_call(
        _tc_c_body,
        grid=(N // _TCR,),
        in_specs=[_rows_spec(), _rows_spec(), _rows_spec(),
                  _full_spec((1, D)), _full_spec((D, D)), _full_spec((1, D)),
                  _rows_spec(DW), _rows_spec(DW)],
        out_specs=_rows_spec(),
        out_shape=jax.ShapeDtypeStruct((N, D), jnp.float32),
    )(p, q, hp, b, W, bl, d0, d1)


@jax.jit
def kernel(x, edge_index, W1, b1, W2, b2, Wl, bl):
    src = edge_index[0].astype(jnp.int32)
    dst = edge_index[1].astype(jnp.int32)
    pad = EPAD - E
    srcp = jnp.concatenate([src, jnp.zeros((pad,), jnp.int32)])
    dstp = jnp.concatenate([dst, jnp.full((pad,), PAD_ROW, jnp.int32)])
    srcp = srcp.reshape(NW, CPT, B)
    dstp = dstp.reshape(NW, CPT, B)

    ones_w = jnp.ones((B, DW), jnp.float32)
    zeros_w = jnp.zeros((RPT, DW), jnp.float32)
    zeros_d = jnp.zeros((ZR2, D), jnp.float32)
    srcp2 = srcp.reshape(TOT_CH, B)
    dstp2 = dstp.reshape(TOT_CH, B)

    degp = _make_deg_sc()(dstp, ones_w, zeros_w)
    d0 = degp[0, :N, :]
    d1 = degp[1, :N, :]

    def agg(hp):
        parts = _make_agg_sc()(hp, srcp2, dstp2, zeros_d)
        return parts[0, :N], parts[1, :N]

    hp1 = _tc_a(x, W1, d0, d1)
    p0, p1 = agg(hp1)
    hp2 = _tc_b(p0, p1, hp1, b1.reshape(1, D), W2, d0, d1)
    q0, q1 = agg(hp2)
    out = _tc_c(q0, q1, hp2, b2.reshape(1, D), Wl,
                bl.reshape(1, D), d0, d1)
    return out
